# baseline (speedup 1.0000x reference)
import functools

import jax
import jax.numpy as jnp
from jax import lax
from jax.experimental import pallas as pl
from jax.experimental.pallas import tpu as pltpu

G = 8
_BN_EPS = 1e-5


def _row_block(n):
    for b in range(2048, 0, -8):
        if n % b == 0:
            return b
    return n


def _bnmm_body(agg_ref, mv_ref, g_ref, b_ref, w_ref, x_ref, h_ref):
    m = mv_ref[0:1, :]
    v = mv_ref[1:2, :]
    x = jnp.maximum((agg_ref[...] - m) / jnp.sqrt(v + _BN_EPS) * g_ref[...] + b_ref[...], 0.0)
    x_ref[...] = x
    h_ref[...] = jnp.dot(x, w_ref[...], preferred_element_type=jnp.float32)


def _gate_body(agg_ref, mv_ref, g3_ref, b3_ref, x1_ref, x2_ref, wg1_ref, bg1_ref,
               wg2_ref, bg2_ref, xo_ref, gate_ref):
    m = mv_ref[0:1, :]
    v = mv_ref[1:2, :]
    x3 = jnp.maximum((agg_ref[...] - m) / jnp.sqrt(v + _BN_EPS) * g3_ref[...] + b3_ref[...], 0.0)
    xo = x1_ref[...] + x2_ref[...] + x3
    xo_ref[...] = xo
    t = jnp.tanh(jnp.dot(xo, wg1_ref[...], preferred_element_type=jnp.float32)
                 + bg1_ref[...])
    gate_ref[...] = jnp.dot(t, wg2_ref[...], preferred_element_type=jnp.float32) + bg2_ref[...]


def _alpha_body(gate_ref, mrow_ref, e_ref):
    e_ref[...] = jnp.exp(gate_ref[...] - mrow_ref[...])


def _cls_body(p_ref, wc1_ref, bc1_ref, gc1_ref, bec1_ref, wc2_ref, bc2_ref,
              gc2_ref, bec2_ref, wc3_ref, bc3_ref, out_ref):
    def bn(x, g, b):
        m = jnp.mean(x, axis=0, keepdims=True)
        v = jnp.mean((x - m) * (x - m), axis=0, keepdims=True)
        return (x - m) / jnp.sqrt(v + _BN_EPS) * g + b

    pooled = p_ref[...]
    h = jnp.dot(pooled, wc1_ref[...], preferred_element_type=jnp.float32) + bc1_ref[...]
    h = jnp.maximum(bn(h, gc1_ref[...], bec1_ref[...]), 0.0)
    h = jnp.dot(h, wc2_ref[...], preferred_element_type=jnp.float32) + bc2_ref[...]
    h = jnp.maximum(bn(h, gc2_ref[...], bec2_ref[...]), 0.0)
    out_ref[...] = jnp.dot(h, wc3_ref[...], preferred_element_type=jnp.float32) + bc3_ref[...]


def kernel(cnn_features, morph_features, cell_edge_index, batch, W1, b1, W2, b2,
           W3, b3, bn1_g, bn1_b, bn2_g, bn2_b, bn3_g, bn3_b, Wg1, bg1, Wg2, bg2,
           Wc1, bc1, gc1, betac1, Wc2, bc2, gc2, betac2, Wc3, bc3):
    n, dcnn = cnn_features.shape
    hdim = W1.shape[1]
    B = _row_block(n)
    grid = n // B

    loop = jnp.arange(n, dtype=cell_edge_index.dtype)
    srcl = jnp.concatenate([cell_edge_index[0], loop])
    dstl = jnp.concatenate([cell_edge_index[1], loop])

    deg = jax.ops.segment_sum(jnp.ones(srcl.shape[0], jnp.float32), dstl, num_segments=n)
    dis = jnp.where(deg > 0, 1.0 / jnp.sqrt(deg), 0.0)
    norm = dis[srcl] * dis[dstl]

    row = lambda i: (i, 0)
    fixed = lambda i: (0, 0)

    x = jnp.concatenate([cnn_features, morph_features], axis=1)
    h1 = x @ W1

    def agg_of(h, bias):
        agg = jax.ops.segment_sum(h[srcl] * norm[:, None], dstl, num_segments=n) + bias
        mv = jnp.stack([agg.mean(axis=0), agg.var(axis=0)])
        return agg, mv

    def bn_mm(agg, mv, g, bb, w):
        return pl.pallas_call(
            _bnmm_body,
            grid=(grid,),
            in_specs=[
                pl.BlockSpec((B, hdim), row),
                pl.BlockSpec((2, hdim), fixed),
                pl.BlockSpec((1, hdim), fixed),
                pl.BlockSpec((1, hdim), fixed),
                pl.BlockSpec((hdim, hdim), fixed),
            ],
            out_specs=[pl.BlockSpec((B, hdim), row), pl.BlockSpec((B, hdim), row)],
            out_shape=[
                jax.ShapeDtypeStruct((n, hdim), jnp.float32),
                jax.ShapeDtypeStruct((n, hdim), jnp.float32),
            ],
        )(agg, mv, g.reshape(1, hdim), bb.reshape(1, hdim), w)

    agg1, mv1 = agg_of(h1, b1)
    x1, h2 = bn_mm(agg1, mv1, bn1_g, bn1_b, W2)
    agg2, mv2 = agg_of(h2, b2)
    x2, h3 = bn_mm(agg2, mv2, bn2_g, bn2_b, W3)
    agg3, mv3 = agg_of(h3, b3)

    xo, gate = pl.pallas_call(
        _gate_body,
        grid=(grid,),
        in_specs=[
            pl.BlockSpec((B, hdim), row),
            pl.BlockSpec((2, hdim), fixed),
            pl.BlockSpec((1, hdim), fixed),
            pl.BlockSpec((1, hdim), fixed),
            pl.BlockSpec((B, hdim), row),
            pl.BlockSpec((B, hdim), row),
            pl.BlockSpec((hdim, hdim), fixed),
            pl.BlockSpec((1, hdim), fixed),
            pl.BlockSpec((hdim, 1), fixed),
            pl.BlockSpec((1, 1), fixed),
        ],
        out_specs=[pl.BlockSpec((B, hdim), row), pl.BlockSpec((B, 1), row)],
        out_shape=[
            jax.ShapeDtypeStruct((n, hdim), jnp.float32),
            jax.ShapeDtypeStruct((n, 1), jnp.float32),
        ],
    )(agg3, mv3, bn3_g.reshape(1, hdim), bn3_b.reshape(1, hdim), x1, x2,
      Wg1, bg1.reshape(1, hdim), Wg2, bg2.reshape(1, 1))

    m = jax.ops.segment_max(gate, batch, num_segments=G)
    e = pl.pallas_call(
        _alpha_body,
        grid=(grid,),
        in_specs=[pl.BlockSpec((B, 1), row), pl.BlockSpec((B, 1), row)],
        out_specs=pl.BlockSpec((B, 1), row),
        out_shape=jax.ShapeDtypeStruct((n, 1), jnp.float32),
    )(gate, m[batch])
    s = jax.ops.segment_sum(e, batch, num_segments=G)
    pooled = jax.ops.segment_sum((e / s[batch]) * xo, batch, num_segments=G)

    h2w = Wc1.shape[1]
    c = Wc3.shape[1]
    out = pl.pallas_call(
        _cls_body,
        out_shape=jax.ShapeDtypeStruct((G, c), jnp.float32),
    )(pooled, Wc1, bc1.reshape(1, h2w), gc1.reshape(1, h2w), betac1.reshape(1, h2w),
      Wc2, bc2.reshape(1, hdim), gc2.reshape(1, hdim), betac2.reshape(1, hdim),
      Wc3, bc3.reshape(1, c))
    return out


# speedup vs baseline: 1.0492x; 1.0492x over previous
import functools

import jax
import jax.numpy as jnp
from jax import lax
from jax.experimental import pallas as pl
from jax.experimental.pallas import tpu as pltpu
from jax.experimental.pallas import tpu_sc as plsc

G = 8
_BN_EPS = 1e-5


def _row_block(n):
    for b in range(2048, 0, -8):
        if n % b == 0:
            return b
    return n


def _bnmm_body(agg_ref, mv_ref, g_ref, b_ref, w_ref, dis_ref, x_ref, h_ref, *, scale):
    m = mv_ref[0:1, :]
    v = mv_ref[1:2, :]
    x = jnp.maximum((agg_ref[...] - m) / jnp.sqrt(v + _BN_EPS) * g_ref[...] + b_ref[...], 0.0)
    x_ref[...] = x
    hh = jnp.dot(x, w_ref[...], preferred_element_type=jnp.float32)
    h_ref[...] = dis_ref[...] * hh if scale else hh


def _gate_body(agg_ref, mv_ref, g3_ref, b3_ref, x1_ref, x2_ref, wg1_ref, bg1_ref,
               wg2_ref, bg2_ref, xo_ref, gate_ref):
    m = mv_ref[0:1, :]
    v = mv_ref[1:2, :]
    x3 = jnp.maximum((agg_ref[...] - m) / jnp.sqrt(v + _BN_EPS) * g3_ref[...] + b3_ref[...], 0.0)
    xo = x1_ref[...] + x2_ref[...] + x3
    xo_ref[...] = xo
    t = jnp.tanh(jnp.dot(xo, wg1_ref[...], preferred_element_type=jnp.float32)
                 + bg1_ref[...])
    gate_ref[...] = jnp.dot(t, wg2_ref[...], preferred_element_type=jnp.float32) + bg2_ref[...]


def _alpha_body(gate_ref, mrow_ref, e_ref):
    e_ref[...] = jnp.exp(gate_ref[...] - mrow_ref[...])


def _cls_body(p_ref, wc1_ref, bc1_ref, gc1_ref, bec1_ref, wc2_ref, bc2_ref,
              gc2_ref, bec2_ref, wc3_ref, bc3_ref, out_ref):
    def bn(x, g, b):
        m = jnp.mean(x, axis=0, keepdims=True)
        v = jnp.mean((x - m) * (x - m), axis=0, keepdims=True)
        return (x - m) / jnp.sqrt(v + _BN_EPS) * g + b

    pooled = p_ref[...]
    h = jnp.dot(pooled, wc1_ref[...], preferred_element_type=jnp.float32) + bc1_ref[...]
    h = jnp.maximum(bn(h, gc1_ref[...], bec1_ref[...]), 0.0)
    h = jnp.dot(h, wc2_ref[...], preferred_element_type=jnp.float32) + bc2_ref[...]
    h = jnp.maximum(bn(h, gc2_ref[...], bec2_ref[...]), 0.0)
    out_ref[...] = jnp.dot(h, wc3_ref[...], preferred_element_type=jnp.float32) + bc3_ref[...]


def _agg3_body(z_ref, y_ref, dis_ref, b_ref, agg_ref):
    agg_ref[...] = dis_ref[...] * (z_ref[...] + y_ref[...]) + b_ref[...]


_SC_NC, _SC_NS = 2, 16
_SC_R = 6272
_SC_NPASS = 4
_SC_CH = 2000
_SC_GR = 80


def _make_sc_scatter(n, e, h):
    NC, NS, R, NPASS, CH, GR = _SC_NC, _SC_NS, _SC_R, _SC_NPASS, _SC_CH, _SC_GR
    NG = CH // GR
    EPT = e // NS
    NCHUNK = EPT // CH
    WB = R // NS
    ZB = (R + 512) // NS
    GARB = R
    mesh = plsc.VectorSubcoreMesh(core_axis_name="c", subcore_axis_name="s")

    @functools.partial(
        pl.kernel, mesh=mesh,
        out_type=jax.ShapeDtypeStruct((NPASS * NC * R, h), jnp.float32),
        scratch_types=[
            pltpu.VMEM((CH,), jnp.int32),
            pltpu.VMEM((CH,), jnp.int32),
            pltpu.VMEM((NG, GR), jnp.int32),
            pltpu.VMEM((GR, h), jnp.float32),
            pltpu.VMEM((WB, h), jnp.float32),
            pltpu.VMEM_SHARED((R + 512, h), jnp.float32),
            pltpu.SemaphoreType.DMA,
        ],
    )
    def sc_scatter(y_hbm, src_hbm, dst_hbm, zero_hbm, z_hbm,
                   dstb, srcb, locb, rows, wbuf, zacc, sem):
        c = lax.axis_index("c")
        s = lax.axis_index("s")
        ebase = s * EPT

        def do_pass(p, _):
            rs = (p * NC + c) * R
            pltpu.sync_copy(zero_hbm, zacc.at[pl.ds(s * ZB, ZB)])
            plsc.subcore_barrier()

            def do_chunk(ci, _):
                base = ebase + ci * CH
                pltpu.sync_copy(dst_hbm.at[pl.ds(base, CH)], dstb)
                pltpu.sync_copy(src_hbm.at[pl.ds(base, CH)], srcb)

                def vecloop(i, _):
                    d = dstb[pl.ds(i * 16, 16)]
                    in_rng = (d >= rs) & (d < rs + R)
                    dloc = jnp.where(in_rng, d - rs, GARB + (d & 511))
                    g = i // (GR // 16)
                    j = lax.rem(i, GR // 16)
                    locb[g, pl.ds(j * 16, 16)] = dloc
                    return 0

                lax.fori_loop(0, CH // 16, vecloop, 0, unroll=5)

                def gloop(g, _):
                    pltpu.async_copy(y_hbm.at[srcb.at[pl.ds(g * GR, GR)]], rows, sem).wait()
                    pltpu.sync_copy(rows, zacc.at[locb.at[g]], add=True)
                    return 0

                lax.fori_loop(0, NG, gloop, 0)
                return 0

            lax.fori_loop(0, NCHUNK, do_chunk, 0)
            plsc.subcore_barrier()
            pltpu.sync_copy(zacc.at[pl.ds(s * WB, WB)], wbuf)
            pltpu.sync_copy(wbuf, z_hbm.at[pl.ds(rs + s * WB, WB)])
            plsc.subcore_barrier()
            return 0

        lax.fori_loop(0, NPASS, do_pass, 0)

    def call(y, src, dst):
        zero = jnp.zeros((ZB, h), jnp.float32)
        return sc_scatter(y, src, dst, zero)[:n]

    return call


def kernel(cnn_features, morph_features, cell_edge_index, batch, W1, b1, W2, b2,
           W3, b3, bn1_g, bn1_b, bn2_g, bn2_b, bn3_g, bn3_b, Wg1, bg1, Wg2, bg2,
           Wc1, bc1, gc1, betac1, Wc2, bc2, gc2, betac2, Wc3, bc3):
    n, dcnn = cnn_features.shape
    hdim = W1.shape[1]
    B = _row_block(n)
    grid = n // B

    loop = jnp.arange(n, dtype=cell_edge_index.dtype)
    srcl = jnp.concatenate([cell_edge_index[0], loop])
    dstl = jnp.concatenate([cell_edge_index[1], loop])

    deg = jax.ops.segment_sum(jnp.ones(srcl.shape[0], jnp.float32), dstl, num_segments=n)
    dis = jnp.where(deg > 0, 1.0 / jnp.sqrt(deg), 0.0)
    norm = dis[srcl] * dis[dstl]

    row = lambda i: (i, 0)
    fixed = lambda i: (0, 0)

    x = jnp.concatenate([cnn_features, morph_features], axis=1)
    h1 = x @ W1

    def agg_of(h, bias):
        agg = jax.ops.segment_sum(h[srcl] * norm[:, None], dstl, num_segments=n) + bias
        mv = jnp.stack([agg.mean(axis=0), agg.var(axis=0)])
        return agg, mv

    dis2 = dis.reshape(n, 1)

    def bn_mm(agg, mv, g, bb, w, scale):
        return pl.pallas_call(
            functools.partial(_bnmm_body, scale=scale),
            grid=(grid,),
            in_specs=[
                pl.BlockSpec((B, hdim), row),
                pl.BlockSpec((2, hdim), fixed),
                pl.BlockSpec((1, hdim), fixed),
                pl.BlockSpec((1, hdim), fixed),
                pl.BlockSpec((hdim, hdim), fixed),
                pl.BlockSpec((B, 1), row),
            ],
            out_specs=[pl.BlockSpec((B, hdim), row), pl.BlockSpec((B, hdim), row)],
            out_shape=[
                jax.ShapeDtypeStruct((n, hdim), jnp.float32),
                jax.ShapeDtypeStruct((n, hdim), jnp.float32),
            ],
        )(agg, mv, g.reshape(1, hdim), bb.reshape(1, hdim), w, dis2)

    agg1, mv1 = agg_of(h1, b1)
    x1, h2 = bn_mm(agg1, mv1, bn1_g, bn1_b, W2, scale=False)
    agg2, mv2 = agg_of(h2, b2)
    x2, y3 = bn_mm(agg2, mv2, bn2_g, bn2_b, W3, scale=True)

    e_edges = cell_edge_index.shape[1]
    z3 = _make_sc_scatter(n, e_edges, hdim)(y3, cell_edge_index[0], cell_edge_index[1])
    agg3 = pl.pallas_call(
        _agg3_body,
        grid=(grid,),
        in_specs=[
            pl.BlockSpec((B, hdim), row),
            pl.BlockSpec((B, hdim), row),
            pl.BlockSpec((B, 1), row),
            pl.BlockSpec((1, hdim), fixed),
        ],
        out_specs=pl.BlockSpec((B, hdim), row),
        out_shape=jax.ShapeDtypeStruct((n, hdim), jnp.float32),
    )(z3, y3, dis2, b3.reshape(1, hdim))
    mv3 = jnp.stack([agg3.mean(axis=0), agg3.var(axis=0)])

    xo, gate = pl.pallas_call(
        _gate_body,
        grid=(grid,),
        in_specs=[
            pl.BlockSpec((B, hdim), row),
            pl.BlockSpec((2, hdim), fixed),
            pl.BlockSpec((1, hdim), fixed),
            pl.BlockSpec((1, hdim), fixed),
            pl.BlockSpec((B, hdim), row),
            pl.BlockSpec((B, hdim), row),
            pl.BlockSpec((hdim, hdim), fixed),
            pl.BlockSpec((1, hdim), fixed),
            pl.BlockSpec((hdim, 1), fixed),
            pl.BlockSpec((1, 1), fixed),
        ],
        out_specs=[pl.BlockSpec((B, hdim), row), pl.BlockSpec((B, 1), row)],
        out_shape=[
            jax.ShapeDtypeStruct((n, hdim), jnp.float32),
            jax.ShapeDtypeStruct((n, 1), jnp.float32),
        ],
    )(agg3, mv3, bn3_g.reshape(1, hdim), bn3_b.reshape(1, hdim), x1, x2,
      Wg1, bg1.reshape(1, hdim), Wg2, bg2.reshape(1, 1))

    m = jax.ops.segment_max(gate, batch, num_segments=G)
    e = pl.pallas_call(
        _alpha_body,
        grid=(grid,),
        in_specs=[pl.BlockSpec((B, 1), row), pl.BlockSpec((B, 1), row)],
        out_specs=pl.BlockSpec((B, 1), row),
        out_shape=jax.ShapeDtypeStruct((n, 1), jnp.float32),
    )(gate, m[batch])
    s = jax.ops.segment_sum(e, batch, num_segments=G)
    pooled = jax.ops.segment_sum((e / s[batch]) * xo, batch, num_segments=G)

    h2w = Wc1.shape[1]
    c = Wc3.shape[1]
    out = pl.pallas_call(
        _cls_body,
        out_shape=jax.ShapeDtypeStruct((G, c), jnp.float32),
    )(pooled, Wc1, bc1.reshape(1, h2w), gc1.reshape(1, h2w), betac1.reshape(1, h2w),
      Wc2, bc2.reshape(1, hdim), gc2.reshape(1, hdim), betac2.reshape(1, hdim),
      Wc3, bc3.reshape(1, c))
    return out


# speedup vs baseline: 1.1021x; 1.0504x over previous
import functools

import jax
import jax.numpy as jnp
from jax import lax
from jax.experimental import pallas as pl
from jax.experimental.pallas import tpu as pltpu
from jax.experimental.pallas import tpu_sc as plsc

G = 8
_BN_EPS = 1e-5


def _row_block(n):
    for b in range(2048, 0, -8):
        if n % b == 0:
            return b
    return n


def _bnmm_body(agg_ref, mv_ref, g_ref, b_ref, w_ref, dis_ref, x_ref, h_ref, *, scale):
    m = mv_ref[0:1, :]
    v = mv_ref[1:2, :]
    x = jnp.maximum((agg_ref[...] - m) / jnp.sqrt(v + _BN_EPS) * g_ref[...] + b_ref[...], 0.0)
    x_ref[...] = x
    hh = jnp.dot(x, w_ref[...], preferred_element_type=jnp.float32)
    h_ref[...] = dis_ref[...] * hh if scale else hh


def _gate_body(agg_ref, mv_ref, g3_ref, b3_ref, x1_ref, x2_ref, wg1_ref, bg1_ref,
               wg2_ref, bg2_ref, xo_ref, gate_ref):
    m = mv_ref[0:1, :]
    v = mv_ref[1:2, :]
    x3 = jnp.maximum((agg_ref[...] - m) / jnp.sqrt(v + _BN_EPS) * g3_ref[...] + b3_ref[...], 0.0)
    xo = x1_ref[...] + x2_ref[...] + x3
    xo_ref[...] = xo
    t = jnp.tanh(jnp.dot(xo, wg1_ref[...], preferred_element_type=jnp.float32)
                 + bg1_ref[...])
    gate_ref[...] = jnp.dot(t, wg2_ref[...], preferred_element_type=jnp.float32) + bg2_ref[...]


def _alpha_body(gate_ref, mrow_ref, e_ref):
    e_ref[...] = jnp.exp(gate_ref[...] - mrow_ref[...])


def _cls_body(p_ref, wc1_ref, bc1_ref, gc1_ref, bec1_ref, wc2_ref, bc2_ref,
              gc2_ref, bec2_ref, wc3_ref, bc3_ref, out_ref):
    def bn(x, g, b):
        m = jnp.mean(x, axis=0, keepdims=True)
        v = jnp.mean((x - m) * (x - m), axis=0, keepdims=True)
        return (x - m) / jnp.sqrt(v + _BN_EPS) * g + b

    pooled = p_ref[...]
    h = jnp.dot(pooled, wc1_ref[...], preferred_element_type=jnp.float32) + bc1_ref[...]
    h = jnp.maximum(bn(h, gc1_ref[...], bec1_ref[...]), 0.0)
    h = jnp.dot(h, wc2_ref[...], preferred_element_type=jnp.float32) + bc2_ref[...]
    h = jnp.maximum(bn(h, gc2_ref[...], bec2_ref[...]), 0.0)
    out_ref[...] = jnp.dot(h, wc3_ref[...], preferred_element_type=jnp.float32) + bc3_ref[...]


def _agg3_body(z_ref, y_ref, dis_ref, b_ref, agg_ref):
    agg_ref[...] = dis_ref[...] * (z_ref[...] + y_ref[...]) + b_ref[...]


_SC_NC, _SC_NS = 2, 16
_SC_R = 6272
_SC_NPASS = 4
_SC_CH = 2048
_SC_GR = 64
_SC_EPT = 51200


def _make_sc_scatter(n, e, h):
    NC, NS, R, NPASS, CH, GR = _SC_NC, _SC_NS, _SC_R, _SC_NPASS, _SC_CH, _SC_GR
    NG = CH // GR
    EPT = _SC_EPT
    EPAD = EPT * NS
    NCHUNK = EPT // CH
    WB = R // NS
    ZB = (R + 512) // NS
    GARB = R
    mesh = plsc.VectorSubcoreMesh(core_axis_name="c", subcore_axis_name="s")

    @functools.partial(
        pl.kernel, mesh=mesh,
        out_type=jax.ShapeDtypeStruct((NPASS * NC * R, h), jnp.float32),
        scratch_types=[
            pltpu.VMEM((CH,), jnp.int32),
            pltpu.VMEM((CH,), jnp.int32),
            pltpu.VMEM((NG, GR), jnp.int32),
            pltpu.VMEM((2 * GR, h), jnp.float32),
            pltpu.VMEM((WB, h), jnp.float32),
            pltpu.VMEM_SHARED((R + 512, h), jnp.float32),
            pltpu.SemaphoreType.DMA,
            pltpu.SemaphoreType.DMA,
        ],
    )
    def sc_scatter(y_hbm, src_hbm, dst_hbm, zero_hbm, z_hbm,
                   dstb, srcb, locb, rows, wbuf, zacc, sem0, sem1):
        c = lax.axis_index("c")
        s = lax.axis_index("s")
        ebase = s * EPT
        sems = (sem0, sem1)

        def do_pass(p, _):
            rs = (p * NC + c) * R
            pltpu.sync_copy(zero_hbm, zacc.at[pl.ds(s * ZB, ZB)])
            plsc.subcore_barrier()

            def do_chunk(ci, _):
                base = ebase + ci * CH
                pltpu.sync_copy(dst_hbm.at[pl.ds(base, CH)], dstb)
                pltpu.sync_copy(src_hbm.at[pl.ds(base, CH)], srcb)

                def vecloop(i, _):
                    d = dstb[pl.ds(i * 16, 16)]
                    in_rng = (d >= rs) & (d < rs + R)
                    dloc = jnp.where(in_rng, d - rs, GARB + (d & 511))
                    g = i // (GR // 16)
                    j = lax.rem(i, GR // 16)
                    locb[g, pl.ds(j * 16, 16)] = dloc
                    return 0

                lax.fori_loop(0, CH // 16, vecloop, 0, unroll=8)

                prev = pltpu.async_copy(
                    y_hbm.at[srcb.at[pl.ds(0, GR)]], rows.at[pl.ds(0, GR)], sems[0])
                for g in range(NG):
                    cur = prev
                    if g + 1 < NG:
                        pb = (g + 1) % 2
                        prev = pltpu.async_copy(
                            y_hbm.at[srcb.at[pl.ds((g + 1) * GR, GR)]],
                            rows.at[pl.ds(pb * GR, GR)], sems[pb])
                    cur.wait()
                    pltpu.sync_copy(rows.at[pl.ds((g % 2) * GR, GR)],
                                    zacc.at[locb.at[g]], add=True)
                return 0

            lax.fori_loop(0, NCHUNK, do_chunk, 0)
            plsc.subcore_barrier()
            pltpu.sync_copy(zacc.at[pl.ds(s * WB, WB)], wbuf)
            pltpu.sync_copy(wbuf, z_hbm.at[pl.ds(rs + s * WB, WB)])
            plsc.subcore_barrier()
            return 0

        lax.fori_loop(0, NPASS, do_pass, 0)

    def call(y, src, dst):
        npad = EPAD - e
        pad_src = (jnp.arange(npad, dtype=src.dtype) * 577) % jnp.int32(n)
        srcp = jnp.concatenate([src, pad_src])
        dstp = jnp.concatenate([dst, jnp.full((npad,), -1, dst.dtype)])
        zero = jnp.zeros((ZB, h), jnp.float32)
        return sc_scatter(y, srcp, dstp, zero)[:n]

    return call


def kernel(cnn_features, morph_features, cell_edge_index, batch, W1, b1, W2, b2,
           W3, b3, bn1_g, bn1_b, bn2_g, bn2_b, bn3_g, bn3_b, Wg1, bg1, Wg2, bg2,
           Wc1, bc1, gc1, betac1, Wc2, bc2, gc2, betac2, Wc3, bc3):
    n, dcnn = cnn_features.shape
    hdim = W1.shape[1]
    B = _row_block(n)
    grid = n // B

    loop = jnp.arange(n, dtype=cell_edge_index.dtype)
    srcl = jnp.concatenate([cell_edge_index[0], loop])
    dstl = jnp.concatenate([cell_edge_index[1], loop])

    deg = jax.ops.segment_sum(jnp.ones(srcl.shape[0], jnp.float32), dstl, num_segments=n)
    dis = jnp.where(deg > 0, 1.0 / jnp.sqrt(deg), 0.0)
    norm = dis[srcl] * dis[dstl]

    row = lambda i: (i, 0)
    fixed = lambda i: (0, 0)

    x = jnp.concatenate([cnn_features, morph_features], axis=1)
    h1 = x @ W1

    def agg_of(h, bias):
        agg = jax.ops.segment_sum(h[srcl] * norm[:, None], dstl, num_segments=n) + bias
        mv = jnp.stack([agg.mean(axis=0), agg.var(axis=0)])
        return agg, mv

    dis2 = dis.reshape(n, 1)

    def bn_mm(agg, mv, g, bb, w, scale):
        return pl.pallas_call(
            functools.partial(_bnmm_body, scale=scale),
            grid=(grid,),
            in_specs=[
                pl.BlockSpec((B, hdim), row),
                pl.BlockSpec((2, hdim), fixed),
                pl.BlockSpec((1, hdim), fixed),
                pl.BlockSpec((1, hdim), fixed),
                pl.BlockSpec((hdim, hdim), fixed),
                pl.BlockSpec((B, 1), row),
            ],
            out_specs=[pl.BlockSpec((B, hdim), row), pl.BlockSpec((B, hdim), row)],
            out_shape=[
                jax.ShapeDtypeStruct((n, hdim), jnp.float32),
                jax.ShapeDtypeStruct((n, hdim), jnp.float32),
            ],
        )(agg, mv, g.reshape(1, hdim), bb.reshape(1, hdim), w, dis2)

    agg1, mv1 = agg_of(h1, b1)
    x1, h2 = bn_mm(agg1, mv1, bn1_g, bn1_b, W2, scale=False)
    agg2, mv2 = agg_of(h2, b2)
    x2, y3 = bn_mm(agg2, mv2, bn2_g, bn2_b, W3, scale=True)

    e_edges = cell_edge_index.shape[1]
    z3 = _make_sc_scatter(n, e_edges, hdim)(y3, cell_edge_index[0], cell_edge_index[1])
    agg3 = pl.pallas_call(
        _agg3_body,
        grid=(grid,),
        in_specs=[
            pl.BlockSpec((B, hdim), row),
            pl.BlockSpec((B, hdim), row),
            pl.BlockSpec((B, 1), row),
            pl.BlockSpec((1, hdim), fixed),
        ],
        out_specs=pl.BlockSpec((B, hdim), row),
        out_shape=jax.ShapeDtypeStruct((n, hdim), jnp.float32),
    )(z3, y3, dis2, b3.reshape(1, hdim))
    mv3 = jnp.stack([agg3.mean(axis=0), agg3.var(axis=0)])

    xo, gate = pl.pallas_call(
        _gate_body,
        grid=(grid,),
        in_specs=[
            pl.BlockSpec((B, hdim), row),
            pl.BlockSpec((2, hdim), fixed),
            pl.BlockSpec((1, hdim), fixed),
            pl.BlockSpec((1, hdim), fixed),
            pl.BlockSpec((B, hdim), row),
            pl.BlockSpec((B, hdim), row),
            pl.BlockSpec((hdim, hdim), fixed),
            pl.BlockSpec((1, hdim), fixed),
            pl.BlockSpec((hdim, 1), fixed),
            pl.BlockSpec((1, 1), fixed),
        ],
        out_specs=[pl.BlockSpec((B, hdim), row), pl.BlockSpec((B, 1), row)],
        out_shape=[
            jax.ShapeDtypeStruct((n, hdim), jnp.float32),
            jax.ShapeDtypeStruct((n, 1), jnp.float32),
        ],
    )(agg3, mv3, bn3_g.reshape(1, hdim), bn3_b.reshape(1, hdim), x1, x2,
      Wg1, bg1.reshape(1, hdim), Wg2, bg2.reshape(1, 1))

    m = jax.ops.segment_max(gate, batch, num_segments=G)
    e = pl.pallas_call(
        _alpha_body,
        grid=(grid,),
        in_specs=[pl.BlockSpec((B, 1), row), pl.BlockSpec((B, 1), row)],
        out_specs=pl.BlockSpec((B, 1), row),
        out_shape=jax.ShapeDtypeStruct((n, 1), jnp.float32),
    )(gate, m[batch])
    s = jax.ops.segment_sum(e, batch, num_segments=G)
    pooled = jax.ops.segment_sum((e / s[batch]) * xo, batch, num_segments=G)

    h2w = Wc1.shape[1]
    c = Wc3.shape[1]
    out = pl.pallas_call(
        _cls_body,
        out_shape=jax.ShapeDtypeStruct((G, c), jnp.float32),
    )(pooled, Wc1, bc1.reshape(1, h2w), gc1.reshape(1, h2w), betac1.reshape(1, h2w),
      Wc2, bc2.reshape(1, hdim), gc2.reshape(1, hdim), betac2.reshape(1, hdim),
      Wc3, bc3.reshape(1, c))
    return out
